# Initial kernel scaffold; baseline (speedup 1.0000x reference)
import jax
import jax.numpy as jnp
from jax.experimental import pallas as pl


def kernel(x, Wr, br, W1, b1, W2, b2):
    raise NotImplementedError("write your pallas kernel here")


# speedup vs baseline: 1.5490x; 1.5490x over previous
import jax
import jax.numpy as jnp
from jax import lax
from jax.experimental import pallas as pl
from jax.experimental.pallas import tpu as pltpu

_D = 768
_DFF = 3072
_E = 8
_TEMP = 0.7
_N = 2048
_FFB = 1024
_NFFB = _DFF // _FFB
_RC = 512


def _moe_body(x_ref, Wr_ref, br_ref, b1_ref, b2_ref, W1_ref, W2_ref,
              out_ref, aux_ref, wd_ref):
    e = pl.program_id(0)
    f = pl.program_id(1)

    @pl.when((e == 0) & (f == 0))
    def _router():
        xf = x_ref[...]
        logits = jnp.dot(xf, Wr_ref[...], preferred_element_type=jnp.float32)
        logits = logits + br_ref[...]
        z = logits * (1.0 / _TEMP)
        z = z - jnp.max(z, axis=-1, keepdims=True)
        ez = jnp.exp(z)
        probs = ez / jnp.sum(ez, axis=-1, keepdims=True)
        iota = lax.broadcasted_iota(jnp.int32, (_N, _E), 1)
        p1 = jnp.max(probs, axis=-1, keepdims=True)
        i1 = jnp.min(jnp.where(probs == p1, iota, _E), axis=-1,
                     keepdims=True)
        probs2 = jnp.where(iota == i1, -1.0, probs)
        p2 = jnp.max(probs2, axis=-1, keepdims=True)
        i2 = jnp.min(jnp.where(probs2 == p2, iota, _E), axis=-1,
                     keepdims=True)
        denom = p1 + p2 + 1e-6
        sel1 = (iota == i1).astype(jnp.float32)
        sel2 = (iota == i2).astype(jnp.float32)
        wd_ref[...] = sel1 * (p1 / denom) + sel2 * (p2 / denom)
        counts = jnp.sum(sel1 + sel2, axis=0)
        pmean = jnp.mean(probs, axis=0)
        aux = _E * jnp.sum(counts * (1.0 / _N) * pmean)
        aux_ref[...] = jnp.full((1, 1), aux, jnp.float32)
        out_ref[...] = jnp.zeros_like(out_ref)

    w1 = W1_ref[0].astype(jnp.bfloat16)
    w2 = W2_ref[0].astype(jnp.bfloat16)
    b1v = b1_ref[0, 0]
    b2v = b2_ref[0]
    fmask = (f == 0).astype(jnp.float32)
    lane = lax.broadcasted_iota(jnp.int32, (_RC, _E), 1)
    for rc in range(_N // _RC):
        rows = pl.ds(rc * _RC, _RC)
        xa = x_ref[rows, :].astype(jnp.bfloat16)
        h = jnp.dot(xa, w1, preferred_element_type=jnp.float32) + b1v
        h = h * (1.0 / (1.0 + jnp.exp(-h)))
        part = jnp.dot(h.astype(jnp.bfloat16), w2,
                       preferred_element_type=jnp.float32)
        we = jnp.sum(wd_ref[rows, :] * (lane == e).astype(jnp.float32),
                     axis=1, keepdims=True)
        out_ref[rows, :] += we * (part + fmask * b2v)


def kernel(x, Wr, br, W1, b1, W2, b2):
    B, L, D = x.shape
    x2 = x.reshape(-1, D)
    out, aux = pl.pallas_call(
        _moe_body,
        grid=(_E, _NFFB),
        in_specs=[
            pl.BlockSpec((_N, _D), lambda e, f: (0, 0)),
            pl.BlockSpec((_D, _E), lambda e, f: (0, 0)),
            pl.BlockSpec((1, _E), lambda e, f: (0, 0)),
            pl.BlockSpec((1, 1, 1, _FFB), lambda e, f: (e, f, 0, 0)),
            pl.BlockSpec((1, 1, _D), lambda e, f: (e, 0, 0)),
            pl.BlockSpec((1, _D, _FFB), lambda e, f: (e, 0, f)),
            pl.BlockSpec((1, _FFB, _D), lambda e, f: (e, f, 0)),
        ],
        out_specs=[
            pl.BlockSpec((_N, _D), lambda e, f: (0, 0)),
            pl.BlockSpec((1, 1), lambda e, f: (0, 0)),
        ],
        out_shape=[
            jax.ShapeDtypeStruct((_N, _D), jnp.float32),
            jax.ShapeDtypeStruct((1, 1), jnp.float32),
        ],
        scratch_shapes=[pltpu.VMEM((_N, _E), jnp.float32)],
        compiler_params=pltpu.CompilerParams(
            dimension_semantics=("arbitrary", "arbitrary")),
    )(x2, Wr, br.reshape(1, _E), b1.reshape(_E, _NFFB, 1, _FFB),
      b2.reshape(_E, 1, _D), W1, W2)
    return out.reshape(B, L, D), aux.reshape(1)


# speedup vs baseline: 1.8238x; 1.1774x over previous
import functools

import jax
import jax.numpy as jnp
from jax import lax
from jax.experimental import pallas as pl
from jax.experimental.pallas import tpu as pltpu
from jax.experimental.pallas import tpu_sc as plsc

_D = 768
_DFF = 3072
_E = 8
_TEMP = 0.7
_N = 2048
_NP = 2 * _N
_T = 256
_SMAX = _NP + _E * _T
_NBLK = _SMAX // _T
_NC = 2
_NS = 16
_NW = _NC * _NS
_PPW = _NP // _NW
_TPW = _N // _NW


def _router_body(x_ref, Wr_ref, br_ref, aux_ref, pos_ref, wrep_ref,
                 gid_ref, act_ref, fid_ref):
    xf = x_ref[...]
    logits = jnp.dot(xf, Wr_ref[...], preferred_element_type=jnp.float32)
    logits = logits + br_ref[...]
    z = logits * (1.0 / _TEMP)
    z = z - jnp.max(z, axis=-1, keepdims=True)
    ez = jnp.exp(z)
    probs = ez / jnp.sum(ez, axis=-1, keepdims=True)
    iota = lax.broadcasted_iota(jnp.int32, (_N, _E), 1)
    p1 = jnp.max(probs, axis=-1, keepdims=True)
    i1 = jnp.min(jnp.where(probs == p1, iota, _E), axis=-1, keepdims=True)
    probs2 = jnp.where(iota == i1, -1.0, probs)
    p2 = jnp.max(probs2, axis=-1, keepdims=True)
    i2 = jnp.min(jnp.where(probs2 == p2, iota, _E), axis=-1, keepdims=True)
    denom = p1 + p2 + 1e-6
    sel1 = (iota == i1).astype(jnp.float32)
    sel2 = (iota == i2).astype(jnp.float32)

    counts_row = jnp.sum(sel1 + sel2, axis=0)
    pmean = jnp.mean(probs, axis=0)
    aux = _E * jnp.sum(counts_row * (1.0 / _N) * pmean)
    aux_ref[...] = jnp.full((1, 1), aux, jnp.float32)

    w1n = p1 / denom
    w2n = p2 / denom
    wcol = jnp.concatenate([w1n, w2n], axis=0)
    wrep_ref[...] = jnp.broadcast_to(wcol, (_NP, 128))

    onehot = jnp.concatenate([sel1, sel2], axis=0)
    cum = onehot
    sh = 1
    while sh < _NP:
        cum = cum + jnp.concatenate(
            [jnp.zeros((sh, _E), jnp.float32), cum[:-sh, :]], axis=0)
        sh *= 2
    rank = cum - onehot

    ones_col = jnp.ones((_NP, 1), jnp.float32)
    counts_col = lax.dot_general(onehot, ones_col, (((0,), (0,)), ((), ())),
                                 preferred_element_type=jnp.float32)
    pc_col = jnp.floor((counts_col + (_T - 1)) * (1.0 / _T)) * _T
    r8 = lax.broadcasted_iota(jnp.int32, (_E, _E), 0)
    c8 = lax.broadcasted_iota(jnp.int32, (_E, _E), 1)
    strict_lo = (c8 < r8).astype(jnp.float32)
    offs_col = jnp.dot(strict_lo, pc_col,
                       preferred_element_type=jnp.float32)
    total = jnp.sum(pc_col)

    offs_row = lax.dot_general(onehot, offs_col, (((1,), (0,)), ((), ())),
                               preferred_element_type=jnp.float32)
    rank_row = jnp.sum(rank * onehot, axis=1, keepdims=True)
    pos_ref[...] = (offs_row + rank_row).astype(jnp.int32)

    nused = total * (1.0 / _T)
    iota_b = lax.broadcasted_iota(jnp.int32, (_E, _NBLK), 1).astype(jnp.float32)
    iota_c = jnp.minimum(iota_b, nused - 1.0)
    starts = offs_col * (1.0 / _T)
    gid = jnp.sum((iota_c >= starts).astype(jnp.float32), axis=0,
                  keepdims=True) - 1.0
    gid_ref[...] = jnp.broadcast_to(gid, (_E, _NBLK)).astype(jnp.int32)
    act_ref[...] = (iota_b < nused).astype(jnp.int32)
    fid_ref[...] = iota_c.astype(jnp.int32)


def _router_call(x2, Wr, br):
    return pl.pallas_call(
        _router_body,
        in_specs=[
            pl.BlockSpec((_N, _D), lambda: (0, 0)),
            pl.BlockSpec((_D, _E), lambda: (0, 0)),
            pl.BlockSpec((1, _E), lambda: (0, 0)),
        ],
        out_specs=[
            pl.BlockSpec((1, 1), lambda: (0, 0)),
            pl.BlockSpec((_NP, 1), lambda: (0, 0)),
            pl.BlockSpec((_NP, 128), lambda: (0, 0)),
            pl.BlockSpec((_E, _NBLK), lambda: (0, 0)),
            pl.BlockSpec((_E, _NBLK), lambda: (0, 0)),
            pl.BlockSpec((_E, _NBLK), lambda: (0, 0)),
        ],
        out_shape=[
            jax.ShapeDtypeStruct((1, 1), jnp.float32),
            jax.ShapeDtypeStruct((_NP, 1), jnp.int32),
            jax.ShapeDtypeStruct((_NP, 128), jnp.float32),
            jax.ShapeDtypeStruct((_E, _NBLK), jnp.int32),
            jax.ShapeDtypeStruct((_E, _NBLK), jnp.int32),
            jax.ShapeDtypeStruct((_E, _NBLK), jnp.int32),
        ],
    )(x2, Wr, br.reshape(1, _E))


def _dispatch_body(x_hbm, wrep_hbm, pos_hbm, xs_hbm, ws_hbm,
                   idx_v, rows_v, wv, sem1, sem2):
    wid = lax.axis_index("s") * _NC + lax.axis_index("c")
    base = wid * _PPW
    n_base = lax.rem(base, _N)
    pltpu.sync_copy(pos_hbm.at[pl.ds(base, _PPW)], idx_v)
    pltpu.sync_copy(x_hbm.at[pl.ds(n_base, _PPW)], rows_v)
    pltpu.sync_copy(wrep_hbm.at[pl.ds(base, _PPW)], wv)
    cp1 = pltpu.async_copy(rows_v, xs_hbm.at[idx_v], sem1)
    cp2 = pltpu.async_copy(wv, ws_hbm.at[idx_v], sem2)
    cp1.wait()
    cp2.wait()


def _dispatch_call(x2, wrep, pos_f):
    mesh = plsc.VectorSubcoreMesh(core_axis_name="c", subcore_axis_name="s",
                                  num_cores=_NC, num_subcores=_NS)
    return pl.kernel(
        _dispatch_body,
        out_type=[
            jax.ShapeDtypeStruct((_SMAX, _D), jnp.float32),
            jax.ShapeDtypeStruct((_SMAX, 128), jnp.float32),
        ],
        mesh=mesh,
        scratch_types=[
            pltpu.VMEM((_PPW,), jnp.int32),
            pltpu.VMEM((_PPW, _D), jnp.float32),
            pltpu.VMEM((_PPW, 128), jnp.float32),
            pltpu.SemaphoreType.DMA,
            pltpu.SemaphoreType.DMA,
        ],
    )(x2, wrep, pos_f)


def _ffn_body(gid_ref, act_ref, fid_ref, xs_ref, ws_ref, W1_ref, W2_ref,
              b1_ref, b2_ref, y_ref):
    b = pl.program_id(0)

    @pl.when(act_ref[b] > 0)
    def _():
        w1 = W1_ref[0].astype(jnp.bfloat16)
        w2 = W2_ref[0].astype(jnp.bfloat16)
        xa = xs_ref[...].astype(jnp.bfloat16)
        h = jnp.dot(xa, w1, preferred_element_type=jnp.float32) + b1_ref[0]
        h = h * (1.0 / (1.0 + jnp.exp(-h)))
        part = jnp.dot(h.astype(jnp.bfloat16), w2,
                       preferred_element_type=jnp.float32) + b2_ref[0]
        y_ref[...] = part * ws_ref[:, :1]


def _ffn_call(gid, act, fid, xs, ws, W1, b1, W2, b2):
    grid_spec = pltpu.PrefetchScalarGridSpec(
        num_scalar_prefetch=3,
        grid=(_NBLK,),
        in_specs=[
            pl.BlockSpec((_T, _D), lambda b, g, a, f: (f[b], 0)),
            pl.BlockSpec((_T, 128), lambda b, g, a, f: (f[b], 0)),
            pl.BlockSpec((1, _D, _DFF), lambda b, g, a, f: (g[b], 0, 0)),
            pl.BlockSpec((1, _DFF, _D), lambda b, g, a, f: (g[b], 0, 0)),
            pl.BlockSpec((1, 1, _DFF), lambda b, g, a, f: (g[b], 0, 0)),
            pl.BlockSpec((1, 1, _D), lambda b, g, a, f: (g[b], 0, 0)),
        ],
        out_specs=pl.BlockSpec((_T, _D), lambda b, g, a, f: (f[b], 0)),
    )
    return pl.pallas_call(
        _ffn_body,
        grid_spec=grid_spec,
        out_shape=jax.ShapeDtypeStruct((_SMAX, _D), jnp.float32),
        compiler_params=pltpu.CompilerParams(
            dimension_semantics=("arbitrary",)),
    )(gid, act, fid, xs, ws, W1, W2, b1.reshape(_E, 1, _DFF),
      b2.reshape(_E, 1, _D))


def _combine_body(y_hbm, pos_hbm, out_hbm, idx0_v, idx1_v, rows0_v, rows1_v,
                  sem0, sem1):
    wid = lax.axis_index("s") * _NC + lax.axis_index("c")
    tbase = wid * _TPW
    pltpu.sync_copy(pos_hbm.at[pl.ds(tbase, _TPW)], idx0_v)
    pltpu.sync_copy(pos_hbm.at[pl.ds(_N + tbase, _TPW)], idx1_v)
    cp0 = pltpu.async_copy(y_hbm.at[idx0_v], rows0_v, sem0)
    cp1 = pltpu.async_copy(y_hbm.at[idx1_v], rows1_v, sem1)
    cp0.wait()
    cp1.wait()

    def body(r, _):
        for j in range(_D // 16):
            c = j * 16
            rows0_v[r, pl.ds(c, 16)] = (rows0_v[r, pl.ds(c, 16)]
                                        + rows1_v[r, pl.ds(c, 16)])
        return _

    lax.fori_loop(0, _TPW, body, None)
    pltpu.sync_copy(rows0_v, out_hbm.at[pl.ds(tbase, _TPW)])


def _combine_call(y, pos_f):
    mesh = plsc.VectorSubcoreMesh(core_axis_name="c", subcore_axis_name="s",
                                  num_cores=_NC, num_subcores=_NS)
    return pl.kernel(
        _combine_body,
        out_type=jax.ShapeDtypeStruct((_N, _D), jnp.float32),
        mesh=mesh,
        scratch_types=[
            pltpu.VMEM((_TPW,), jnp.int32),
            pltpu.VMEM((_TPW,), jnp.int32),
            pltpu.VMEM((_TPW, _D), jnp.float32),
            pltpu.VMEM((_TPW, _D), jnp.float32),
            pltpu.SemaphoreType.DMA,
            pltpu.SemaphoreType.DMA,
        ],
    )(y, pos_f)


def kernel(x, Wr, br, W1, b1, W2, b2):
    B, L, D = x.shape
    x2 = x.reshape(-1, D)
    aux, pos, wrep, gid, act, fid = _router_call(x2, Wr, br)
    pos_f = pos.reshape(_NP)
    xs, ws = _dispatch_call(x2, wrep, pos_f)
    y = _ffn_call(gid[0], act[0], fid[0], xs, ws, W1, b1, W2, b2)
    out = _combine_call(y, pos_f)
    return out.reshape(B, L, D), aux.reshape(1)


# speedup vs baseline: 2.1037x; 1.1535x over previous
import functools

import jax
import jax.numpy as jnp
from jax import lax
from jax.experimental import pallas as pl
from jax.experimental.pallas import tpu as pltpu
from jax.experimental.pallas import tpu_sc as plsc

_D = 768
_DFF = 3072
_E = 8
_TEMP = 0.7
_N = 2048
_NP = 2 * _N
_T = 256
_SMAX = _NP + _E * _T
_NBLK = _SMAX // _T
_NC = 2
_NS = 16
_NW = _NC * _NS
_PPW = _NP // _NW
_TPW = _N // _NW


def _router_body(x_ref, Wr_ref, br_ref, aux_ref, pos_ref, wrep_ref,
                 gid_ref, act_ref, fid_ref, meta_ref):
    xf = x_ref[...]
    logits = jnp.dot(xf, Wr_ref[...], preferred_element_type=jnp.float32)
    logits = logits + br_ref[...]
    z = logits * (1.0 / _TEMP)
    z = z - jnp.max(z, axis=-1, keepdims=True)
    ez = jnp.exp(z)
    probs = ez / jnp.sum(ez, axis=-1, keepdims=True)
    iota = lax.broadcasted_iota(jnp.int32, (_N, _E), 1)
    p1 = jnp.max(probs, axis=-1, keepdims=True)
    i1 = jnp.min(jnp.where(probs == p1, iota, _E), axis=-1, keepdims=True)
    probs2 = jnp.where(iota == i1, -1.0, probs)
    p2 = jnp.max(probs2, axis=-1, keepdims=True)
    i2 = jnp.min(jnp.where(probs2 == p2, iota, _E), axis=-1, keepdims=True)
    denom = p1 + p2 + 1e-6
    sel1 = (iota == i1).astype(jnp.float32)
    sel2 = (iota == i2).astype(jnp.float32)

    counts_row = jnp.sum(sel1 + sel2, axis=0)
    pmean = jnp.mean(probs, axis=0)
    aux = _E * jnp.sum(counts_row * (1.0 / _N) * pmean)
    aux_ref[...] = jnp.full((1, 1), aux, jnp.float32)

    w1n = p1 / denom
    w2n = p2 / denom
    wcol = jnp.concatenate([w1n, w2n], axis=0)
    wrep_ref[...] = jnp.broadcast_to(wcol, (_NP, 128))

    onehot = jnp.concatenate([sel1, sel2], axis=0)
    cum = onehot
    sh = 1
    while sh < _NP:
        cum = cum + jnp.concatenate(
            [jnp.zeros((sh, _E), jnp.float32), cum[:-sh, :]], axis=0)
        sh *= 2
    rank = cum - onehot

    ones_col = jnp.ones((_NP, 1), jnp.float32)
    counts_col = lax.dot_general(onehot, ones_col, (((0,), (0,)), ((), ())),
                                 preferred_element_type=jnp.float32)
    pc_col = jnp.floor((counts_col + (_T - 1)) * (1.0 / _T)) * _T
    r8 = lax.broadcasted_iota(jnp.int32, (_E, _E), 0)
    c8 = lax.broadcasted_iota(jnp.int32, (_E, _E), 1)
    strict_lo = (c8 < r8).astype(jnp.float32)
    offs_col = jnp.dot(strict_lo, pc_col,
                       preferred_element_type=jnp.float32)
    total = jnp.sum(pc_col)

    offs_row = lax.dot_general(onehot, offs_col, (((1,), (0,)), ((), ())),
                               preferred_element_type=jnp.float32)
    rank_row = jnp.sum(rank * onehot, axis=1, keepdims=True)
    pos_ref[...] = (offs_row + rank_row).astype(jnp.int32)

    nused = total * (1.0 / _T)
    iota_b = lax.broadcasted_iota(jnp.int32, (_E, _NBLK), 1).astype(jnp.float32)
    iota_c = jnp.minimum(iota_b, nused - 1.0)
    starts = offs_col * (1.0 / _T)
    gid = jnp.sum((iota_c >= starts).astype(jnp.float32), axis=0,
                  keepdims=True) - 1.0
    gid_ref[...] = jnp.broadcast_to(gid, (_E, _NBLK)).astype(jnp.int32)
    act_ref[...] = (iota_b < nused).astype(jnp.int32)
    fid_ref[...] = iota_c.astype(jnp.int32)

    def _mod2(v):
        return v - 2.0 * jnp.floor(v * 0.5)

    used_col = (counts_col > 0).astype(jnp.float32)
    ordc_col = jnp.dot(strict_lo, used_col,
                       preferred_element_type=jnp.float32)
    nu = jnp.sum(used_col)
    usedf = used_col * (iota_c >= starts).astype(jnp.float32)
    ord_b = jnp.sum(usedf, axis=0, keepdims=True) - 1.0
    par_b = _mod2(ord_b)
    first_b = jnp.sum(used_col * (starts == iota_b).astype(jnp.float32),
                      axis=0, keepdims=True)
    o1 = ord_b + 1.0
    hasnxt = (o1 <= nu - 1.0).astype(jnp.float32)
    o1c = jnp.minimum(o1, nu - 1.0)
    e_col = lax.broadcasted_iota(jnp.int32, (_E, 1), 0).astype(jnp.float32)
    nxt1 = jnp.sum(e_col * used_col * (ordc_col == o1c).astype(jnp.float32),
                   axis=0, keepdims=True)
    meta = jnp.concatenate([
        gid,
        (iota_b[:1] < nused).astype(jnp.float32),
        iota_c[:1],
        par_b, first_b, nxt1, hasnxt,
        jnp.zeros((1, _NBLK), jnp.float32)], axis=0)
    meta_ref[...] = meta.astype(jnp.int32)


def _router_call(x2, Wr, br):
    return pl.pallas_call(
        _router_body,
        in_specs=[
            pl.BlockSpec((_N, _D), lambda: (0, 0)),
            pl.BlockSpec((_D, _E), lambda: (0, 0)),
            pl.BlockSpec((1, _E), lambda: (0, 0)),
        ],
        out_specs=[
            pl.BlockSpec((1, 1), lambda: (0, 0)),
            pl.BlockSpec((_NP, 1), lambda: (0, 0)),
            pl.BlockSpec((_NP, 128), lambda: (0, 0)),
            pl.BlockSpec((_E, _NBLK), lambda: (0, 0)),
            pl.BlockSpec((_E, _NBLK), lambda: (0, 0)),
            pl.BlockSpec((_E, _NBLK), lambda: (0, 0)),
            pl.BlockSpec((8, _NBLK), lambda: (0, 0)),
        ],
        out_shape=[
            jax.ShapeDtypeStruct((1, 1), jnp.float32),
            jax.ShapeDtypeStruct((_NP, 1), jnp.int32),
            jax.ShapeDtypeStruct((_NP, 128), jnp.float32),
            jax.ShapeDtypeStruct((_E, _NBLK), jnp.int32),
            jax.ShapeDtypeStruct((_E, _NBLK), jnp.int32),
            jax.ShapeDtypeStruct((_E, _NBLK), jnp.int32),
            jax.ShapeDtypeStruct((8, _NBLK), jnp.int32),
        ],
    )(x2, Wr, br.reshape(1, _E))


def _dispatch_body(x_hbm, wrep_hbm, pos_hbm, xs_hbm, ws_hbm,
                   idx_v, rows_v, wv, sem1, sem2):
    wid = lax.axis_index("s") * _NC + lax.axis_index("c")
    base = wid * _PPW
    n_base = lax.rem(base, _N)
    pltpu.sync_copy(pos_hbm.at[pl.ds(base, _PPW)], idx_v)
    pltpu.sync_copy(x_hbm.at[pl.ds(n_base, _PPW)], rows_v)
    pltpu.sync_copy(wrep_hbm.at[pl.ds(base, _PPW)], wv)
    cp1 = pltpu.async_copy(rows_v, xs_hbm.at[idx_v], sem1)
    cp2 = pltpu.async_copy(wv, ws_hbm.at[idx_v], sem2)
    cp1.wait()
    cp2.wait()


def _dispatch_call(x2, wrep, pos_f):
    mesh = plsc.VectorSubcoreMesh(core_axis_name="c", subcore_axis_name="s",
                                  num_cores=_NC, num_subcores=_NS)
    return pl.kernel(
        _dispatch_body,
        out_type=[
            jax.ShapeDtypeStruct((_SMAX, _D), jnp.float32),
            jax.ShapeDtypeStruct((_SMAX, 128), jnp.float32),
        ],
        mesh=mesh,
        scratch_types=[
            pltpu.VMEM((_PPW,), jnp.int32),
            pltpu.VMEM((_PPW, _D), jnp.float32),
            pltpu.VMEM((_PPW, 128), jnp.float32),
            pltpu.SemaphoreType.DMA,
            pltpu.SemaphoreType.DMA,
        ],
    )(x2, wrep, pos_f)


def _ffn_body(meta_ref, xs_ref, ws_ref, W1_hbm, W2_hbm, b1_ref, b2_ref,
              y_ref, w1a, w1b, w2a, w2b, s1a, s1b, s2a, s2b):
    b = pl.program_id(0)
    gid = meta_ref[0, b]
    act = meta_ref[1, b]
    par = meta_ref[3, b]
    first = meta_ref[4, b]
    nxt1 = meta_ref[5, b]
    hasnxt = meta_ref[6, b]

    @pl.when(b == 0)
    def _():
        pltpu.make_async_copy(W1_hbm.at[gid], w1a, s1a).start()
        pltpu.make_async_copy(W2_hbm.at[gid], w2a, s2a).start()

    @pl.when((first == 1) & (hasnxt == 1))
    def _():
        @pl.when(par == 0)
        def _():
            pltpu.make_async_copy(W1_hbm.at[nxt1], w1b, s1b).start()
            pltpu.make_async_copy(W2_hbm.at[nxt1], w2b, s2b).start()

        @pl.when(par == 1)
        def _():
            pltpu.make_async_copy(W1_hbm.at[nxt1], w1a, s1a).start()
            pltpu.make_async_copy(W2_hbm.at[nxt1], w2a, s2a).start()

    @pl.when((first == 1) & (par == 0))
    def _():
        pltpu.make_async_copy(W1_hbm.at[gid], w1a, s1a).wait()
        pltpu.make_async_copy(W2_hbm.at[gid], w2a, s2a).wait()

    @pl.when((first == 1) & (par == 1))
    def _():
        pltpu.make_async_copy(W1_hbm.at[gid], w1b, s1b).wait()
        pltpu.make_async_copy(W2_hbm.at[gid], w2b, s2b).wait()

    def _compute(w1_ref, w2_ref):
        w1 = w1_ref[...].astype(jnp.bfloat16)
        w2 = w2_ref[...].astype(jnp.bfloat16)
        xa = xs_ref[...].astype(jnp.bfloat16)
        h = jnp.dot(xa, w1, preferred_element_type=jnp.float32) + b1_ref[0]
        h = h * (1.0 / (1.0 + jnp.exp(-h)))
        part = jnp.dot(h.astype(jnp.bfloat16), w2,
                       preferred_element_type=jnp.float32) + b2_ref[0]
        y_ref[...] = part * ws_ref[:, :1]

    @pl.when((act == 1) & (par == 0))
    def _():
        _compute(w1a, w2a)

    @pl.when((act == 1) & (par == 1))
    def _():
        _compute(w1b, w2b)


def _ffn_call(meta, xs, ws, W1, b1, W2, b2):
    grid_spec = pltpu.PrefetchScalarGridSpec(
        num_scalar_prefetch=1,
        grid=(_NBLK,),
        in_specs=[
            pl.BlockSpec((_T, _D), lambda b, m: (m[2, b], 0)),
            pl.BlockSpec((_T, 128), lambda b, m: (m[2, b], 0)),
            pl.BlockSpec(memory_space=pl.ANY),
            pl.BlockSpec(memory_space=pl.ANY),
            pl.BlockSpec((1, 1, _DFF), lambda b, m: (m[0, b], 0, 0)),
            pl.BlockSpec((1, 1, _D), lambda b, m: (m[0, b], 0, 0)),
        ],
        out_specs=pl.BlockSpec((_T, _D), lambda b, m: (m[2, b], 0)),
        scratch_shapes=[
            pltpu.VMEM((_D, _DFF), jnp.float32),
            pltpu.VMEM((_D, _DFF), jnp.float32),
            pltpu.VMEM((_DFF, _D), jnp.float32),
            pltpu.VMEM((_DFF, _D), jnp.float32),
            pltpu.SemaphoreType.DMA,
            pltpu.SemaphoreType.DMA,
            pltpu.SemaphoreType.DMA,
            pltpu.SemaphoreType.DMA,
        ],
    )
    return pl.pallas_call(
        _ffn_body,
        grid_spec=grid_spec,
        out_shape=jax.ShapeDtypeStruct((_SMAX, _D), jnp.float32),
        compiler_params=pltpu.CompilerParams(
            dimension_semantics=("arbitrary",)),
    )(meta, xs, ws, W1, W2, b1.reshape(_E, 1, _DFF), b2.reshape(_E, 1, _D))


def _combine_body(y_hbm, pos_hbm, out_hbm, idx0_v, idx1_v, rows0_v, rows1_v,
                  sem0, sem1):
    wid = lax.axis_index("s") * _NC + lax.axis_index("c")
    tbase = wid * _TPW
    pltpu.sync_copy(pos_hbm.at[pl.ds(tbase, _TPW)], idx0_v)
    pltpu.sync_copy(pos_hbm.at[pl.ds(_N + tbase, _TPW)], idx1_v)
    cp0 = pltpu.async_copy(y_hbm.at[idx0_v], rows0_v, sem0)
    cp1 = pltpu.async_copy(y_hbm.at[idx1_v], rows1_v, sem1)
    cp0.wait()
    cp1.wait()

    def body(r, _):
        for j in range(_D // 16):
            c = j * 16
            rows0_v[r, pl.ds(c, 16)] = (rows0_v[r, pl.ds(c, 16)]
                                        + rows1_v[r, pl.ds(c, 16)])
        return _

    lax.fori_loop(0, _TPW, body, None)
    pltpu.sync_copy(rows0_v, out_hbm.at[pl.ds(tbase, _TPW)])


def _combine_call(y, pos_f):
    mesh = plsc.VectorSubcoreMesh(core_axis_name="c", subcore_axis_name="s",
                                  num_cores=_NC, num_subcores=_NS)
    return pl.kernel(
        _combine_body,
        out_type=jax.ShapeDtypeStruct((_N, _D), jnp.float32),
        mesh=mesh,
        scratch_types=[
            pltpu.VMEM((_TPW,), jnp.int32),
            pltpu.VMEM((_TPW,), jnp.int32),
            pltpu.VMEM((_TPW, _D), jnp.float32),
            pltpu.VMEM((_TPW, _D), jnp.float32),
            pltpu.SemaphoreType.DMA,
            pltpu.SemaphoreType.DMA,
        ],
    )(y, pos_f)


def kernel(x, Wr, br, W1, b1, W2, b2):
    B, L, D = x.shape
    x2 = x.reshape(-1, D)
    aux, pos, wrep, gid, act, fid, meta = _router_call(x2, Wr, br)
    pos_f = pos.reshape(_NP)
    xs, ws = _dispatch_call(x2, wrep, pos_f)
    y = _ffn_call(meta, xs, ws, W1, b1, W2, b2)
    out = _combine_call(y, pos_f)
    return out.reshape(B, L, D), aux.reshape(1)


# speedup vs baseline: 2.1449x; 1.0196x over previous
import functools

import jax
import jax.numpy as jnp
from jax import lax
from jax.experimental import pallas as pl
from jax.experimental.pallas import tpu as pltpu
from jax.experimental.pallas import tpu_sc as plsc

_D = 768
_DFF = 3072
_E = 8
_TEMP = 0.7
_N = 2048
_NP = 2 * _N
_T = 256
_SMAX = _NP + _E * _T
_NBLK = _SMAX // _T
_DH = _D // 2
_NC = 2
_NS = 16
_NW = _NC * _NS
_PPW = _NP // _NW
_TPW = _N // _NW


def _router_body(x_ref, Wr_ref, br_ref, aux_ref, pos_ref, wrep_ref,
                 gid_ref, act_ref, fid_ref, meta_ref, xb_ref):
    xf = x_ref[...]
    logits = jnp.dot(xf, Wr_ref[...], preferred_element_type=jnp.float32)
    logits = logits + br_ref[...]
    z = logits * (1.0 / _TEMP)
    z = z - jnp.max(z, axis=-1, keepdims=True)
    ez = jnp.exp(z)
    probs = ez / jnp.sum(ez, axis=-1, keepdims=True)
    iota = lax.broadcasted_iota(jnp.int32, (_N, _E), 1)
    p1 = jnp.max(probs, axis=-1, keepdims=True)
    i1 = jnp.min(jnp.where(probs == p1, iota, _E), axis=-1, keepdims=True)
    probs2 = jnp.where(iota == i1, -1.0, probs)
    p2 = jnp.max(probs2, axis=-1, keepdims=True)
    i2 = jnp.min(jnp.where(probs2 == p2, iota, _E), axis=-1, keepdims=True)
    denom = p1 + p2 + 1e-6
    sel1 = (iota == i1).astype(jnp.float32)
    sel2 = (iota == i2).astype(jnp.float32)

    counts_row = jnp.sum(sel1 + sel2, axis=0)
    pmean = jnp.mean(probs, axis=0)
    aux = _E * jnp.sum(counts_row * (1.0 / _N) * pmean)
    aux_ref[...] = jnp.full((1, 1), aux, jnp.float32)

    xbits = lax.bitcast_convert_type(xf, jnp.int32)
    rlo = lax.shift_right_logical(xbits[:, :_DH] + 0x8000, 16)
    rhi = (xbits[:, _DH:] + 0x8000) & jnp.int32(-65536)
    xb_ref[...] = rhi | rlo

    w1n = p1 / denom
    w2n = p2 / denom
    wcol = jnp.concatenate([w1n, w2n], axis=0)
    wrep_ref[...] = jnp.broadcast_to(wcol, (_NP, 128))

    onehot = jnp.concatenate([sel1, sel2], axis=0)
    cum = onehot
    sh = 1
    while sh < _NP:
        cum = cum + jnp.concatenate(
            [jnp.zeros((sh, _E), jnp.float32), cum[:-sh, :]], axis=0)
        sh *= 2
    rank = cum - onehot

    ones_col = jnp.ones((_NP, 1), jnp.float32)
    counts_col = lax.dot_general(onehot, ones_col, (((0,), (0,)), ((), ())),
                                 preferred_element_type=jnp.float32)
    pc_col = jnp.floor((counts_col + (_T - 1)) * (1.0 / _T)) * _T
    r8 = lax.broadcasted_iota(jnp.int32, (_E, _E), 0)
    c8 = lax.broadcasted_iota(jnp.int32, (_E, _E), 1)
    strict_lo = (c8 < r8).astype(jnp.float32)
    offs_col = jnp.dot(strict_lo, pc_col,
                       preferred_element_type=jnp.float32)
    total = jnp.sum(pc_col)

    offs_row = lax.dot_general(onehot, offs_col, (((1,), (0,)), ((), ())),
                               preferred_element_type=jnp.float32)
    rank_row = jnp.sum(rank * onehot, axis=1, keepdims=True)
    pos_ref[...] = (offs_row + rank_row).astype(jnp.int32)

    nused = total * (1.0 / _T)
    iota_b = lax.broadcasted_iota(jnp.int32, (_E, _NBLK), 1).astype(jnp.float32)
    iota_c = jnp.minimum(iota_b, nused - 1.0)
    starts = offs_col * (1.0 / _T)
    gid = jnp.sum((iota_c >= starts).astype(jnp.float32), axis=0,
                  keepdims=True) - 1.0
    gid_ref[...] = jnp.broadcast_to(gid, (_E, _NBLK)).astype(jnp.int32)
    act_ref[...] = (iota_b < nused).astype(jnp.int32)
    fid_ref[...] = iota_c.astype(jnp.int32)

    def _mod2(v):
        return v - 2.0 * jnp.floor(v * 0.5)

    used_col = (counts_col > 0).astype(jnp.float32)
    ordc_col = jnp.dot(strict_lo, used_col,
                       preferred_element_type=jnp.float32)
    nu = jnp.sum(used_col)
    usedf = used_col * (iota_c >= starts).astype(jnp.float32)
    ord_b = jnp.sum(usedf, axis=0, keepdims=True) - 1.0
    par_b = _mod2(ord_b)
    first_b = jnp.sum(used_col * (starts == iota_b).astype(jnp.float32),
                      axis=0, keepdims=True)
    o1 = ord_b + 1.0
    hasnxt = (o1 <= nu - 1.0).astype(jnp.float32)
    o1c = jnp.minimum(o1, nu - 1.0)
    e_col = lax.broadcasted_iota(jnp.int32, (_E, 1), 0).astype(jnp.float32)
    nxt1 = jnp.sum(e_col * used_col * (ordc_col == o1c).astype(jnp.float32),
                   axis=0, keepdims=True)
    meta = jnp.concatenate([
        gid,
        (iota_b[:1] < nused).astype(jnp.float32),
        iota_c[:1],
        par_b, first_b, nxt1, hasnxt,
        jnp.zeros((1, _NBLK), jnp.float32)], axis=0)
    meta_ref[...] = meta.astype(jnp.int32)


def _router_call(x2, Wr, br):
    return pl.pallas_call(
        _router_body,
        in_specs=[
            pl.BlockSpec((_N, _D), lambda: (0, 0)),
            pl.BlockSpec((_D, _E), lambda: (0, 0)),
            pl.BlockSpec((1, _E), lambda: (0, 0)),
        ],
        out_specs=[
            pl.BlockSpec((1, 1), lambda: (0, 0)),
            pl.BlockSpec((_NP, 1), lambda: (0, 0)),
            pl.BlockSpec((_NP, 128), lambda: (0, 0)),
            pl.BlockSpec((_E, _NBLK), lambda: (0, 0)),
            pl.BlockSpec((_E, _NBLK), lambda: (0, 0)),
            pl.BlockSpec((_E, _NBLK), lambda: (0, 0)),
            pl.BlockSpec((8, _NBLK), lambda: (0, 0)),
            pl.BlockSpec((_N, _DH), lambda: (0, 0)),
        ],
        out_shape=[
            jax.ShapeDtypeStruct((1, 1), jnp.float32),
            jax.ShapeDtypeStruct((_NP, 1), jnp.int32),
            jax.ShapeDtypeStruct((_NP, 128), jnp.float32),
            jax.ShapeDtypeStruct((_E, _NBLK), jnp.int32),
            jax.ShapeDtypeStruct((_E, _NBLK), jnp.int32),
            jax.ShapeDtypeStruct((_E, _NBLK), jnp.int32),
            jax.ShapeDtypeStruct((8, _NBLK), jnp.int32),
            jax.ShapeDtypeStruct((_N, _DH), jnp.int32),
        ],
    )(x2, Wr, br.reshape(1, _E))


def _dispatch_body(x_hbm, wrep_hbm, pos_hbm, xs_hbm, ws_hbm,
                   idx_v, rows_v, wv, sem1, sem2):
    wid = lax.axis_index("s") * _NC + lax.axis_index("c")
    base = wid * _PPW
    n_base = lax.rem(base, _N)
    pltpu.sync_copy(pos_hbm.at[pl.ds(base, _PPW)], idx_v)
    pltpu.sync_copy(x_hbm.at[pl.ds(n_base, _PPW)], rows_v)
    pltpu.sync_copy(wrep_hbm.at[pl.ds(base, _PPW)], wv)
    cp1 = pltpu.async_copy(rows_v, xs_hbm.at[idx_v], sem1)
    cp2 = pltpu.async_copy(wv, ws_hbm.at[idx_v], sem2)
    cp1.wait()
    cp2.wait()


def _dispatch_call(x2, wrep, pos_f):
    mesh = plsc.VectorSubcoreMesh(core_axis_name="c", subcore_axis_name="s",
                                  num_cores=_NC, num_subcores=_NS)
    return pl.kernel(
        _dispatch_body,
        out_type=[
            jax.ShapeDtypeStruct((_SMAX, _DH), jnp.int32),
            jax.ShapeDtypeStruct((_SMAX, 128), jnp.float32),
        ],
        mesh=mesh,
        scratch_types=[
            pltpu.VMEM((_PPW,), jnp.int32),
            pltpu.VMEM((_PPW, _DH), jnp.int32),
            pltpu.VMEM((_PPW, 128), jnp.float32),
            pltpu.SemaphoreType.DMA,
            pltpu.SemaphoreType.DMA,
        ],
    )(x2, wrep, pos_f)


def _ffn_body(meta_ref, xs_ref, ws_ref, W1_hbm, W2_hbm, b1_ref, b2_ref,
              y_ref, w1a, w1b, w2a, w2b, s1a, s1b, s2a, s2b):
    b = pl.program_id(0)
    gid = meta_ref[0, b]
    act = meta_ref[1, b]
    par = meta_ref[3, b]
    first = meta_ref[4, b]
    nxt1 = meta_ref[5, b]
    hasnxt = meta_ref[6, b]

    @pl.when(b == 0)
    def _():
        pltpu.make_async_copy(W1_hbm.at[gid], w1a, s1a).start()
        pltpu.make_async_copy(W2_hbm.at[gid], w2a, s2a).start()

    @pl.when((first == 1) & (hasnxt == 1))
    def _():
        @pl.when(par == 0)
        def _():
            pltpu.make_async_copy(W1_hbm.at[nxt1], w1b, s1b).start()
            pltpu.make_async_copy(W2_hbm.at[nxt1], w2b, s2b).start()

        @pl.when(par == 1)
        def _():
            pltpu.make_async_copy(W1_hbm.at[nxt1], w1a, s1a).start()
            pltpu.make_async_copy(W2_hbm.at[nxt1], w2a, s2a).start()

    @pl.when((first == 1) & (par == 0))
    def _():
        pltpu.make_async_copy(W1_hbm.at[gid], w1a, s1a).wait()
        pltpu.make_async_copy(W2_hbm.at[gid], w2a, s2a).wait()

    @pl.when((first == 1) & (par == 1))
    def _():
        pltpu.make_async_copy(W1_hbm.at[gid], w1b, s1b).wait()
        pltpu.make_async_copy(W2_hbm.at[gid], w2b, s2b).wait()

    def _compute(w1_ref, w2_ref):
        w1 = w1_ref[...].astype(jnp.bfloat16)
        w2 = w2_ref[...].astype(jnp.bfloat16)
        v = xs_ref[...]
        xlo = lax.bitcast_convert_type(lax.shift_left(v, 16), jnp.float32)
        xhi = lax.bitcast_convert_type(v & jnp.int32(-65536), jnp.float32)
        xa = jnp.concatenate([xlo, xhi], axis=1).astype(jnp.bfloat16)
        h = jnp.dot(xa, w1, preferred_element_type=jnp.float32) + b1_ref[0]
        h = h * (1.0 / (1.0 + jnp.exp(-h)))
        part = jnp.dot(h.astype(jnp.bfloat16), w2,
                       preferred_element_type=jnp.float32) + b2_ref[0]
        part = part * ws_ref[:, :1]
        pbits = lax.bitcast_convert_type(part, jnp.int32)
        plo = lax.shift_right_logical(pbits[:, :_DH] + 0x8000, 16)
        phi = (pbits[:, _DH:] + 0x8000) & jnp.int32(-65536)
        y_ref[...] = phi | plo

    @pl.when((act == 1) & (par == 0))
    def _():
        _compute(w1a, w2a)

    @pl.when((act == 1) & (par == 1))
    def _():
        _compute(w1b, w2b)


def _ffn_call(meta, xs, ws, W1, b1, W2, b2):
    grid_spec = pltpu.PrefetchScalarGridSpec(
        num_scalar_prefetch=1,
        grid=(_NBLK,),
        in_specs=[
            pl.BlockSpec((_T, _DH), lambda b, m: (m[2, b], 0)),
            pl.BlockSpec((_T, 128), lambda b, m: (m[2, b], 0)),
            pl.BlockSpec(memory_space=pl.ANY),
            pl.BlockSpec(memory_space=pl.ANY),
            pl.BlockSpec((1, 1, _DFF), lambda b, m: (m[0, b], 0, 0)),
            pl.BlockSpec((1, 1, _D), lambda b, m: (m[0, b], 0, 0)),
        ],
        out_specs=pl.BlockSpec((_T, _DH), lambda b, m: (m[2, b], 0)),
        scratch_shapes=[
            pltpu.VMEM((_D, _DFF), jnp.float32),
            pltpu.VMEM((_D, _DFF), jnp.float32),
            pltpu.VMEM((_DFF, _D), jnp.float32),
            pltpu.VMEM((_DFF, _D), jnp.float32),
            pltpu.SemaphoreType.DMA,
            pltpu.SemaphoreType.DMA,
            pltpu.SemaphoreType.DMA,
            pltpu.SemaphoreType.DMA,
        ],
    )
    return pl.pallas_call(
        _ffn_body,
        grid_spec=grid_spec,
        out_shape=jax.ShapeDtypeStruct((_SMAX, _DH), jnp.int32),
        compiler_params=pltpu.CompilerParams(
            dimension_semantics=("arbitrary",)),
    )(meta, xs, ws, W1, W2, b1.reshape(_E, 1, _DFF), b2.reshape(_E, 1, _D))


def _combine_body(y_hbm, pos_hbm, out_hbm, idx0_v, idx1_v, rows0_v, rows1_v,
                  out_v, sem0, sem1):
    wid = lax.axis_index("s") * _NC + lax.axis_index("c")
    tbase = wid * _TPW
    pltpu.sync_copy(pos_hbm.at[pl.ds(tbase, _TPW)], idx0_v)
    pltpu.sync_copy(pos_hbm.at[pl.ds(_N + tbase, _TPW)], idx1_v)
    cp0 = pltpu.async_copy(y_hbm.at[idx0_v], rows0_v, sem0)
    cp1 = pltpu.async_copy(y_hbm.at[idx1_v], rows1_v, sem1)
    cp0.wait()
    cp1.wait()
    himask = jnp.full((16,), -65536, jnp.int32)

    def body(r, _):
        for j in range(_DH // 16):
            c = j * 16
            v0 = rows0_v[r, pl.ds(c, 16)]
            v1 = rows1_v[r, pl.ds(c, 16)]
            lo = (lax.bitcast_convert_type(lax.shift_left(v0, 16), jnp.float32)
                  + lax.bitcast_convert_type(lax.shift_left(v1, 16),
                                             jnp.float32))
            hi = (lax.bitcast_convert_type(v0 & himask, jnp.float32)
                  + lax.bitcast_convert_type(v1 & himask, jnp.float32))
            out_v[r, pl.ds(c, 16)] = lo
            out_v[r, pl.ds(_DH + c, 16)] = hi
        return _

    lax.fori_loop(0, _TPW, body, None)
    pltpu.sync_copy(out_v, out_hbm.at[pl.ds(tbase, _TPW)])


def _combine_call(y, pos_f):
    mesh = plsc.VectorSubcoreMesh(core_axis_name="c", subcore_axis_name="s",
                                  num_cores=_NC, num_subcores=_NS)
    return pl.kernel(
        _combine_body,
        out_type=jax.ShapeDtypeStruct((_N, _D), jnp.float32),
        mesh=mesh,
        scratch_types=[
            pltpu.VMEM((_TPW,), jnp.int32),
            pltpu.VMEM((_TPW,), jnp.int32),
            pltpu.VMEM((_TPW, _DH), jnp.int32),
            pltpu.VMEM((_TPW, _DH), jnp.int32),
            pltpu.VMEM((_TPW, _D), jnp.float32),
            pltpu.SemaphoreType.DMA,
            pltpu.SemaphoreType.DMA,
        ],
    )(y, pos_f)


def kernel(x, Wr, br, W1, b1, W2, b2):
    B, L, D = x.shape
    x2 = x.reshape(-1, D)
    aux, pos, wrep, gid, act, fid, meta, xb = _router_call(x2, Wr, br)
    pos_f = pos.reshape(_NP)
    xs, ws = _dispatch_call(xb, wrep, pos_f)
    y = _ffn_call(meta, xs, ws, W1, b1, W2, b2)
    out = _combine_call(y, pos_f)
    return out.reshape(B, L, D), aux.reshape(1)


# speedup vs baseline: 2.2790x; 1.0625x over previous
import functools

import jax
import jax.numpy as jnp
from jax import lax
from jax.experimental import pallas as pl
from jax.experimental.pallas import tpu as pltpu
from jax.experimental.pallas import tpu_sc as plsc

_D = 768
_DFF = 3072
_E = 8
_TEMP = 0.7
_N = 2048
_NP = 2 * _N
_T = 256
_SMAX = _NP + _E * _T
_NBLK = _SMAX // _T
_DH = _D // 2
_NC = 2
_NS = 16
_NW = _NC * _NS
_PPW = _NP // _NW
_TPW = _N // _NW


def _router_body(x_ref, Wr_ref, br_ref, aux_ref, pos_ref, wrep_ref,
                 gid_ref, act_ref, fid_ref, meta_ref, xb_ref):
    xf = x_ref[...]
    logits = jnp.dot(xf, Wr_ref[...], preferred_element_type=jnp.float32)
    logits = logits + br_ref[...]
    z = logits * (1.0 / _TEMP)
    z = z - jnp.max(z, axis=-1, keepdims=True)
    ez = jnp.exp(z)
    probs = ez / jnp.sum(ez, axis=-1, keepdims=True)
    iota = lax.broadcasted_iota(jnp.int32, (_N, _E), 1)
    p1 = jnp.max(probs, axis=-1, keepdims=True)
    i1 = jnp.min(jnp.where(probs == p1, iota, _E), axis=-1, keepdims=True)
    probs2 = jnp.where(iota == i1, -1.0, probs)
    p2 = jnp.max(probs2, axis=-1, keepdims=True)
    i2 = jnp.min(jnp.where(probs2 == p2, iota, _E), axis=-1, keepdims=True)
    denom = p1 + p2 + 1e-6
    sel1 = (iota == i1).astype(jnp.float32)
    sel2 = (iota == i2).astype(jnp.float32)

    counts_row = jnp.sum(sel1 + sel2, axis=0)
    pmean = jnp.mean(probs, axis=0)
    aux = _E * jnp.sum(counts_row * (1.0 / _N) * pmean)
    aux_ref[...] = jnp.full((1, 1), aux, jnp.float32)

    xbits = lax.bitcast_convert_type(xf, jnp.int32)
    rlo = lax.shift_right_logical(xbits[:, :_DH] + 0x8000, 16)
    rhi = (xbits[:, _DH:] + 0x8000) & jnp.int32(-65536)
    xb_ref[...] = rhi | rlo

    w1n = p1 / denom
    w2n = p2 / denom
    wcol = jnp.concatenate([w1n, w2n], axis=0)
    wrep_ref[...] = jnp.broadcast_to(wcol, (_NP, 128))

    S = jnp.concatenate([sel1, sel2], axis=1)
    cum = S
    sh = 1
    while sh < _N:
        cum = cum + jnp.concatenate(
            [jnp.zeros((sh, 2 * _E), jnp.float32), cum[:-sh, :]], axis=0)
        sh *= 2
    exS = cum - S
    last = cum[_N - 1:_N, :]
    c1row = last[:, :_E]
    rank1 = exS[:, :_E]
    rank2 = exS[:, _E:] + c1row

    rq = lax.broadcasted_iota(jnp.int32, (_E, 2 * _E), 0)
    cq = lax.broadcasted_iota(jnp.int32, (_E, 2 * _E), 1)
    Q = ((cq == rq) | (cq == rq + _E)).astype(jnp.float32)
    counts_col = lax.dot_general(Q, last, (((1,), (1,)), ((), ())),
                                 preferred_element_type=jnp.float32)
    pc_col = jnp.floor((counts_col + (_T - 1)) * (1.0 / _T)) * _T
    r8 = lax.broadcasted_iota(jnp.int32, (_E, _E), 0)
    c8 = lax.broadcasted_iota(jnp.int32, (_E, _E), 1)
    strict_lo = (c8 < r8).astype(jnp.float32)
    offs_col = jnp.dot(strict_lo, pc_col,
                       preferred_element_type=jnp.float32)
    total = jnp.sum(pc_col)

    po1 = lax.dot_general(sel1, offs_col, (((1,), (0,)), ((), ())),
                          preferred_element_type=jnp.float32)
    po2 = lax.dot_general(sel2, offs_col, (((1,), (0,)), ((), ())),
                          preferred_element_type=jnp.float32)
    pr1 = jnp.sum(rank1 * sel1, axis=1, keepdims=True)
    pr2 = jnp.sum(rank2 * sel2, axis=1, keepdims=True)
    pos_ref[...] = jnp.concatenate([po1 + pr1, po2 + pr2],
                                   axis=0).astype(jnp.int32)

    nused = total * (1.0 / _T)
    iota_b = lax.broadcasted_iota(jnp.int32, (_E, _NBLK), 1).astype(jnp.float32)
    iota_c = jnp.minimum(iota_b, nused - 1.0)
    starts = offs_col * (1.0 / _T)
    gid = jnp.sum((iota_c >= starts).astype(jnp.float32), axis=0,
                  keepdims=True) - 1.0
    gid_ref[...] = jnp.broadcast_to(gid, (_E, _NBLK)).astype(jnp.int32)
    act_ref[...] = (iota_b < nused).astype(jnp.int32)
    fid_ref[...] = iota_c.astype(jnp.int32)

    def _mod2(v):
        return v - 2.0 * jnp.floor(v * 0.5)

    used_col = (counts_col > 0).astype(jnp.float32)
    ordc_col = jnp.dot(strict_lo, used_col,
                       preferred_element_type=jnp.float32)
    nu = jnp.sum(used_col)
    usedf = used_col * (iota_c >= starts).astype(jnp.float32)
    ord_b = jnp.sum(usedf, axis=0, keepdims=True) - 1.0
    par_b = _mod2(ord_b)
    first_b = jnp.sum(used_col * (starts == iota_b).astype(jnp.float32),
                      axis=0, keepdims=True)
    o1 = ord_b + 1.0
    hasnxt = (o1 <= nu - 1.0).astype(jnp.float32)
    o1c = jnp.minimum(o1, nu - 1.0)
    e_col = lax.broadcasted_iota(jnp.int32, (_E, 1), 0).astype(jnp.float32)
    nxt1 = jnp.sum(e_col * used_col * (ordc_col == o1c).astype(jnp.float32),
                   axis=0, keepdims=True)
    meta = jnp.concatenate([
        gid,
        (iota_b[:1] < nused).astype(jnp.float32),
        iota_c[:1],
        par_b, first_b, nxt1, hasnxt,
        jnp.zeros((1, _NBLK), jnp.float32)], axis=0)
    meta_ref[...] = meta.astype(jnp.int32)


def _router_call(x2, Wr, br):
    return pl.pallas_call(
        _router_body,
        in_specs=[
            pl.BlockSpec((_N, _D), lambda: (0, 0)),
            pl.BlockSpec((_D, _E), lambda: (0, 0)),
            pl.BlockSpec((1, _E), lambda: (0, 0)),
        ],
        out_specs=[
            pl.BlockSpec((1, 1), lambda: (0, 0)),
            pl.BlockSpec((_NP, 1), lambda: (0, 0)),
            pl.BlockSpec((_NP, 128), lambda: (0, 0)),
            pl.BlockSpec((_E, _NBLK), lambda: (0, 0)),
            pl.BlockSpec((_E, _NBLK), lambda: (0, 0)),
            pl.BlockSpec((_E, _NBLK), lambda: (0, 0)),
            pl.BlockSpec((8, _NBLK), lambda: (0, 0)),
            pl.BlockSpec((_N, _DH), lambda: (0, 0)),
        ],
        out_shape=[
            jax.ShapeDtypeStruct((1, 1), jnp.float32),
            jax.ShapeDtypeStruct((_NP, 1), jnp.int32),
            jax.ShapeDtypeStruct((_NP, 128), jnp.float32),
            jax.ShapeDtypeStruct((_E, _NBLK), jnp.int32),
            jax.ShapeDtypeStruct((_E, _NBLK), jnp.int32),
            jax.ShapeDtypeStruct((_E, _NBLK), jnp.int32),
            jax.ShapeDtypeStruct((8, _NBLK), jnp.int32),
            jax.ShapeDtypeStruct((_N, _DH), jnp.int32),
        ],
    )(x2, Wr, br.reshape(1, _E))


def _dispatch_body(x_hbm, wrep_hbm, pos_hbm, xs_hbm, ws_hbm,
                   idx_v, rows_v, wv, sem1, sem2):
    wid = lax.axis_index("s") * _NC + lax.axis_index("c")
    base = wid * _PPW
    n_base = lax.rem(base, _N)
    pltpu.sync_copy(pos_hbm.at[pl.ds(base, _PPW)], idx_v)
    pltpu.sync_copy(x_hbm.at[pl.ds(n_base, _PPW)], rows_v)
    pltpu.sync_copy(wrep_hbm.at[pl.ds(base, _PPW)], wv)
    cp1 = pltpu.async_copy(rows_v, xs_hbm.at[idx_v], sem1)
    cp2 = pltpu.async_copy(wv, ws_hbm.at[idx_v], sem2)
    cp1.wait()
    cp2.wait()


def _dispatch_call(x2, wrep, pos_f):
    mesh = plsc.VectorSubcoreMesh(core_axis_name="c", subcore_axis_name="s",
                                  num_cores=_NC, num_subcores=_NS)
    return pl.kernel(
        _dispatch_body,
        out_type=[
            jax.ShapeDtypeStruct((_SMAX, _DH), jnp.int32),
            jax.ShapeDtypeStruct((_SMAX, 128), jnp.float32),
        ],
        mesh=mesh,
        scratch_types=[
            pltpu.VMEM((_PPW,), jnp.int32),
            pltpu.VMEM((_PPW, _DH), jnp.int32),
            pltpu.VMEM((_PPW, 128), jnp.float32),
            pltpu.SemaphoreType.DMA,
            pltpu.SemaphoreType.DMA,
        ],
    )(x2, wrep, pos_f)


def _ffn_body(meta_ref, xs_ref, ws_ref, W1_hbm, W2_hbm, b1_ref, b2_ref,
              y_ref, w1a, w1b, w2a, w2b, s1a, s1b, s2a, s2b):
    b = pl.program_id(0)
    gid = meta_ref[0, b]
    act = meta_ref[1, b]
    par = meta_ref[3, b]
    first = meta_ref[4, b]
    nxt1 = meta_ref[5, b]
    hasnxt = meta_ref[6, b]

    @pl.when(b == 0)
    def _():
        pltpu.make_async_copy(W1_hbm.at[gid], w1a, s1a).start()
        pltpu.make_async_copy(W2_hbm.at[gid], w2a, s2a).start()

    @pl.when((first == 1) & (hasnxt == 1))
    def _():
        @pl.when(par == 0)
        def _():
            pltpu.make_async_copy(W1_hbm.at[nxt1], w1b, s1b).start()
            pltpu.make_async_copy(W2_hbm.at[nxt1], w2b, s2b).start()

        @pl.when(par == 1)
        def _():
            pltpu.make_async_copy(W1_hbm.at[nxt1], w1a, s1a).start()
            pltpu.make_async_copy(W2_hbm.at[nxt1], w2a, s2a).start()

    @pl.when((first == 1) & (par == 0))
    def _():
        pltpu.make_async_copy(W1_hbm.at[gid], w1a, s1a).wait()
        pltpu.make_async_copy(W2_hbm.at[gid], w2a, s2a).wait()

    @pl.when((first == 1) & (par == 1))
    def _():
        pltpu.make_async_copy(W1_hbm.at[gid], w1b, s1b).wait()
        pltpu.make_async_copy(W2_hbm.at[gid], w2b, s2b).wait()

    def _compute(w1_ref, w2_ref):
        w1 = w1_ref[...].astype(jnp.bfloat16)
        w2 = w2_ref[...].astype(jnp.bfloat16)
        v = xs_ref[...]
        xlo = lax.bitcast_convert_type(lax.shift_left(v, 16), jnp.float32)
        xhi = lax.bitcast_convert_type(v & jnp.int32(-65536), jnp.float32)
        xa = jnp.concatenate([xlo, xhi], axis=1).astype(jnp.bfloat16)
        h = jnp.dot(xa, w1, preferred_element_type=jnp.float32) + b1_ref[0]
        h = h * (1.0 / (1.0 + jnp.exp(-h)))
        part = jnp.dot(h.astype(jnp.bfloat16), w2,
                       preferred_element_type=jnp.float32) + b2_ref[0]
        part = part * ws_ref[:, :1]
        pbits = lax.bitcast_convert_type(part, jnp.int32)
        plo = lax.shift_right_logical(pbits[:, :_DH] + 0x8000, 16)
        phi = (pbits[:, _DH:] + 0x8000) & jnp.int32(-65536)
        y_ref[...] = phi | plo

    @pl.when((act == 1) & (par == 0))
    def _():
        _compute(w1a, w2a)

    @pl.when((act == 1) & (par == 1))
    def _():
        _compute(w1b, w2b)


def _ffn_call(meta, xs, ws, W1, b1, W2, b2):
    grid_spec = pltpu.PrefetchScalarGridSpec(
        num_scalar_prefetch=1,
        grid=(_NBLK,),
        in_specs=[
            pl.BlockSpec((_T, _DH), lambda b, m: (m[2, b], 0)),
            pl.BlockSpec((_T, 128), lambda b, m: (m[2, b], 0)),
            pl.BlockSpec(memory_space=pl.ANY),
            pl.BlockSpec(memory_space=pl.ANY),
            pl.BlockSpec((1, 1, _DFF), lambda b, m: (m[0, b], 0, 0)),
            pl.BlockSpec((1, 1, _D), lambda b, m: (m[0, b], 0, 0)),
        ],
        out_specs=pl.BlockSpec((_T, _DH), lambda b, m: (m[2, b], 0)),
        scratch_shapes=[
            pltpu.VMEM((_D, _DFF), jnp.float32),
            pltpu.VMEM((_D, _DFF), jnp.float32),
            pltpu.VMEM((_DFF, _D), jnp.float32),
            pltpu.VMEM((_DFF, _D), jnp.float32),
            pltpu.SemaphoreType.DMA,
            pltpu.SemaphoreType.DMA,
            pltpu.SemaphoreType.DMA,
            pltpu.SemaphoreType.DMA,
        ],
    )
    return pl.pallas_call(
        _ffn_body,
        grid_spec=grid_spec,
        out_shape=jax.ShapeDtypeStruct((_SMAX, _DH), jnp.int32),
        compiler_params=pltpu.CompilerParams(
            dimension_semantics=("arbitrary",)),
    )(meta, xs, ws, W1, W2, b1.reshape(_E, 1, _DFF), b2.reshape(_E, 1, _D))


def _combine_body(y_hbm, pos_hbm, out_hbm, idx0_v, idx1_v, rows0_v, rows1_v,
                  out_v, sem0, sem1):
    wid = lax.axis_index("s") * _NC + lax.axis_index("c")
    tbase = wid * _TPW
    pltpu.sync_copy(pos_hbm.at[pl.ds(tbase, _TPW)], idx0_v)
    pltpu.sync_copy(pos_hbm.at[pl.ds(_N + tbase, _TPW)], idx1_v)
    cp0 = pltpu.async_copy(y_hbm.at[idx0_v], rows0_v, sem0)
    cp1 = pltpu.async_copy(y_hbm.at[idx1_v], rows1_v, sem1)
    cp0.wait()
    cp1.wait()
    himask = jnp.full((16,), -65536, jnp.int32)

    @plsc.parallel_loop(0, _TPW, unroll=4)
    def _loop(r):
        for j in range(_DH // 16):
            c = j * 16
            v0 = rows0_v[r, pl.ds(c, 16)]
            v1 = rows1_v[r, pl.ds(c, 16)]
            lo = (lax.bitcast_convert_type(lax.shift_left(v0, 16), jnp.float32)
                  + lax.bitcast_convert_type(lax.shift_left(v1, 16),
                                             jnp.float32))
            hi = (lax.bitcast_convert_type(v0 & himask, jnp.float32)
                  + lax.bitcast_convert_type(v1 & himask, jnp.float32))
            out_v[r, pl.ds(c, 16)] = lo
            out_v[r, pl.ds(_DH + c, 16)] = hi

    pltpu.sync_copy(out_v, out_hbm.at[pl.ds(tbase, _TPW)])


def _combine_call(y, pos_f):
    mesh = plsc.VectorSubcoreMesh(core_axis_name="c", subcore_axis_name="s",
                                  num_cores=_NC, num_subcores=_NS)
    return pl.kernel(
        _combine_body,
        out_type=jax.ShapeDtypeStruct((_N, _D), jnp.float32),
        mesh=mesh,
        scratch_types=[
            pltpu.VMEM((_TPW,), jnp.int32),
            pltpu.VMEM((_TPW,), jnp.int32),
            pltpu.VMEM((_TPW, _DH), jnp.int32),
            pltpu.VMEM((_TPW, _DH), jnp.int32),
            pltpu.VMEM((_TPW, _D), jnp.float32),
            pltpu.SemaphoreType.DMA,
            pltpu.SemaphoreType.DMA,
        ],
    )(y, pos_f)


def kernel(x, Wr, br, W1, b1, W2, b2):
    B, L, D = x.shape
    x2 = x.reshape(-1, D)
    aux, pos, wrep, gid, act, fid, meta, xb = _router_call(x2, Wr, br)
    pos_f = pos.reshape(_NP)
    xs, ws = _dispatch_call(xb, wrep, pos_f)
    y = _ffn_call(meta, xs, ws, W1, b1, W2, b2)
    out = _combine_call(y, pos_f)
    return out.reshape(B, L, D), aux.reshape(1)
